# Initial kernel scaffold; baseline (speedup 1.0000x reference)
import jax
import jax.numpy as jnp
from jax.experimental import pallas as pl


def kernel(x, n_adjs, scores):
    raise NotImplementedError("write your pallas kernel here")


# speedup vs baseline: 1.2027x; 1.2027x over previous
import jax
import jax.numpy as jnp
from jax.experimental import pallas as pl
from jax.experimental.pallas import tpu as pltpu

_N = 4096
_BR = 256

_ROT = ((13, 15, 26, 6), (17, 29, 16, 24))


def _threefry_bits(cnt):
    ks = (jnp.uint32(0), jnp.uint32(42), jnp.uint32(0 ^ 42 ^ 0x1BD11BDA))
    x0 = jnp.full(cnt.shape, ks[0], jnp.uint32)
    x1 = cnt + ks[1]
    for i in range(5):
        for r in _ROT[i % 2]:
            x0 = x0 + x1
            x1 = (x1 << jnp.uint32(r)) | (x1 >> jnp.uint32(32 - r))
            x1 = x1 ^ x0
        x0 = x0 + ks[(i + 1) % 3]
        x1 = x1 + ks[(i + 2) % 3] + jnp.uint32(i + 1)
    return x0 ^ x1


def _body(scale_ref, scores_ref, ei_ref, samples_ref):
    i = pl.program_id(0)
    row = jax.lax.broadcasted_iota(jnp.int32, (_BR, _N), 0) + i * _BR
    col = jax.lax.broadcasted_iota(jnp.int32, (_BR, _N), 1)
    cnt = (row * _N + col).astype(jnp.uint32)
    bits = _threefry_bits(cnt)
    mant = (bits >> jnp.uint32(9)) | jnp.uint32(0x3F800000)
    u = jax.lax.bitcast_convert_type(mant, jnp.float32) - jnp.float32(1.0)
    p = jax.nn.sigmoid(scores_ref[...])
    scale = scale_ref[0, 0]
    samples_ref[...] = jnp.where(u < p, scale, jnp.float32(0.0))
    ei_ref[0] = row
    ei_ref[1] = col


def kernel(x, n_adjs, scores):
    del x
    scale = jnp.asarray(n_adjs, jnp.float32).reshape(1, 1)
    ei3, samples = pl.pallas_call(
        _body,
        grid=(_N // _BR,),
        in_specs=[
            pl.BlockSpec(memory_space=pltpu.SMEM),
            pl.BlockSpec((_BR, _N), lambda i: (i, 0)),
        ],
        out_specs=[
            pl.BlockSpec((2, _BR, _N), lambda i: (0, i, 0)),
            pl.BlockSpec((_BR, _N), lambda i: (i, 0)),
        ],
        out_shape=[
            jax.ShapeDtypeStruct((2, _N, _N), jnp.int32),
            jax.ShapeDtypeStruct((_N, _N), jnp.float32),
        ],
        compiler_params=pltpu.CompilerParams(
            dimension_semantics=("arbitrary",),
        ),
    )(scale, scores)
    return (ei3.reshape(2, _N * _N), samples.reshape(_N * _N))


# speedup vs baseline: 1.3005x; 1.0813x over previous
import jax
import jax.numpy as jnp
from jax.experimental import pallas as pl
from jax.experimental.pallas import tpu as pltpu

_N = 4096
_BR = 256
_FS = _BR * _N
_FR = _FS // 128

_ROT = ((13, 15, 26, 6), (17, 29, 16, 24))


def _threefry_bits(cnt):
    ks = (jnp.uint32(0), jnp.uint32(42), jnp.uint32(0 ^ 42 ^ 0x1BD11BDA))
    x0 = jnp.full(cnt.shape, ks[0], jnp.uint32)
    x1 = cnt + ks[1]
    for i in range(5):
        for r in _ROT[i % 2]:
            x0 = x0 + x1
            x1 = (x1 << jnp.uint32(r)) | (x1 >> jnp.uint32(32 - r))
            x1 = x1 ^ x0
        x0 = x0 + ks[(i + 1) % 3]
        x1 = x1 + ks[(i + 2) % 3] + jnp.uint32(i + 1)
    return x0 ^ x1


def _body(scale_ref, scores_ref, ei_ref, samples_ref):
    i = pl.program_id(0)
    row = jax.lax.broadcasted_iota(jnp.int32, (_BR, _N), 0) + i * _BR
    col = jax.lax.broadcasted_iota(jnp.int32, (_BR, _N), 1)
    cnt = (row * _N + col).astype(jnp.uint32)
    bits = _threefry_bits(cnt)
    mant = (bits >> jnp.uint32(9)) | jnp.uint32(0x3F800000)
    u = jax.lax.bitcast_convert_type(mant, jnp.float32) - jnp.float32(1.0)
    p = jax.nn.sigmoid(scores_ref[...])
    scale = scale_ref[0, 0]
    vals = jnp.where(u < p, scale, jnp.float32(0.0))
    samples_ref[...] = vals.reshape(_FR, 128)
    fr = jax.lax.broadcasted_iota(jnp.int32, (_FR, 128), 0)
    fc = jax.lax.broadcasted_iota(jnp.int32, (_FR, 128), 1)
    f = i * _FS + fr * 128 + fc
    ei_ref[0] = f >> 12
    ei_ref[1] = f & (_N - 1)


def kernel(x, n_adjs, scores):
    del x
    scale = jnp.asarray(n_adjs, jnp.float32).reshape(1, 1)
    ei3, samples = pl.pallas_call(
        _body,
        grid=(_N // _BR,),
        in_specs=[
            pl.BlockSpec(memory_space=pltpu.SMEM),
            pl.BlockSpec((_BR, _N), lambda i: (i, 0)),
        ],
        out_specs=[
            pl.BlockSpec((2, _FR, 128), lambda i: (0, i, 0)),
            pl.BlockSpec((_FR, 128), lambda i: (i, 0)),
        ],
        out_shape=[
            jax.ShapeDtypeStruct((2, _N * _N // 128, 128), jnp.int32),
            jax.ShapeDtypeStruct((_N * _N // 128, 128), jnp.float32),
        ],
        compiler_params=pltpu.CompilerParams(
            dimension_semantics=("arbitrary",),
        ),
    )(scale, scores)
    return (ei3.reshape(2, _N * _N), samples.reshape(_N * _N))


# speedup vs baseline: 1.5626x; 1.2016x over previous
import jax
import jax.numpy as jnp
from jax.experimental import pallas as pl
from jax.experimental.pallas import tpu as pltpu

_N = 4096
_BR = 256
_FS = _BR * _N
_FR = _FS // 128

_ROT = ((13, 15, 26, 6), (17, 29, 16, 24))


def _threefry_bits(cnt):
    ks = (jnp.uint32(0), jnp.uint32(42), jnp.uint32(0 ^ 42 ^ 0x1BD11BDA))
    x0 = jnp.full(cnt.shape, ks[0], jnp.uint32)
    x1 = cnt + ks[1]
    for i in range(5):
        for r in _ROT[i % 2]:
            x0 = x0 + x1
            x1 = (x1 << jnp.uint32(r)) | (x1 >> jnp.uint32(32 - r))
            x1 = x1 ^ x0
        x0 = x0 + ks[(i + 1) % 3]
        x1 = x1 + ks[(i + 2) % 3] + jnp.uint32(i + 1)
    return x0 ^ x1


def _ei_body(ei_ref):
    i = pl.program_id(0)
    fr = jax.lax.broadcasted_iota(jnp.int32, (_FR, 128), 0)
    fc = jax.lax.broadcasted_iota(jnp.int32, (_FR, 128), 1)
    f = i * _FS + fr * 128 + fc
    ei_ref[0] = f >> 12
    ei_ref[1] = f & (_N - 1)


def _samples_body(scale_ref, scores_ref, samples_ref):
    i = pl.program_id(0)
    row = jax.lax.broadcasted_iota(jnp.int32, (_BR, _N), 0) + i * _BR
    col = jax.lax.broadcasted_iota(jnp.int32, (_BR, _N), 1)
    cnt = (row * _N + col).astype(jnp.uint32)
    bits = _threefry_bits(cnt)
    mant = (bits >> jnp.uint32(9)) | jnp.uint32(0x3F800000)
    u = jax.lax.bitcast_convert_type(mant, jnp.float32) - jnp.float32(1.0)
    p = jax.nn.sigmoid(scores_ref[...])
    scale = scale_ref[0, 0]
    vals = jnp.where(u < p, scale, jnp.float32(0.0))
    samples_ref[...] = vals.reshape(_FR, 128)


def kernel(x, n_adjs, scores):
    del x
    scale = jnp.asarray(n_adjs, jnp.float32).reshape(1, 1)
    ei3 = pl.pallas_call(
        _ei_body,
        grid=(_N // _BR,),
        out_specs=pl.BlockSpec((2, _FR, 128), lambda i: (0, i, 0)),
        out_shape=jax.ShapeDtypeStruct((2, _N * _N // 128, 128), jnp.int32),
        compiler_params=pltpu.CompilerParams(
            dimension_semantics=("arbitrary",),
        ),
    )()
    samples = pl.pallas_call(
        _samples_body,
        grid=(_N // _BR,),
        in_specs=[
            pl.BlockSpec(memory_space=pltpu.SMEM),
            pl.BlockSpec((_BR, _N), lambda i: (i, 0)),
        ],
        out_specs=pl.BlockSpec((_FR, 128), lambda i: (i, 0)),
        out_shape=jax.ShapeDtypeStruct((_N * _N // 128, 128), jnp.float32),
        compiler_params=pltpu.CompilerParams(
            dimension_semantics=("arbitrary",),
        ),
    )(scale, scores)
    return (ei3.reshape(2, _N * _N), samples.reshape(_N * _N))
